# baseline (speedup 1.0000x reference)
import functools

import jax
import jax.numpy as jnp
from jax import lax
from jax.experimental import pallas as pl
from jax.experimental.pallas import tpu as pltpu
from jax.experimental.pallas import tpu_sc as plsc

F32 = jnp.float32

_NC = 2
_NS = 16
_NW = _NC * _NS
_K = 128


def _pad_nodes(n):
    unit = _NS * 8
    return ((n + 1 + unit - 1) // unit) * unit


def _worker_id():
    return lax.axis_index("s") * _NC + lax.axis_index("c")


def _core_id():
    return lax.axis_index("c")


def _subcore_id():
    return lax.axis_index("s")


def _make_deg_kernel(n_pad, d, nchunk):
    rows_per_sub = n_pad // _NS
    mesh = plsc.VectorSubcoreMesh(core_axis_name="c", subcore_axis_name="s")

    @functools.partial(
        pl.kernel,
        out_type=jax.ShapeDtypeStruct((_NC, n_pad, d), F32),
        mesh=mesh,
        scratch_types=[
            pltpu.VMEM((nchunk, _K), jnp.int32),
            pltpu.VMEM((_K, d), F32),
            pltpu.VMEM_SHARED((n_pad, d), F32),
        ],
    )
    def deg_kernel(dst3_hbm, ones_hbm, zeros_hbm, out_hbm, didx_v, ones_v,
                   acc):
        c = _core_id()
        s = _subcore_id()
        wid = _worker_id()
        r0 = s * rows_per_sub
        pltpu.sync_copy(zeros_hbm.at[pl.ds(r0, rows_per_sub)],
                        acc.at[pl.ds(r0, rows_per_sub)])
        pltpu.sync_copy(dst3_hbm.at[wid], didx_v)
        pltpu.sync_copy(ones_hbm, ones_v)
        plsc.subcore_barrier()

        def body(g, _):
            pltpu.sync_copy(ones_v, acc.at[didx_v.at[g]], add=True)
            return 0

        lax.fori_loop(0, nchunk, body, 0)
        plsc.subcore_barrier()
        pltpu.sync_copy(acc.at[pl.ds(r0, rows_per_sub)],
                        out_hbm.at[c, pl.ds(r0, rows_per_sub)])

    return deg_kernel


_IDXD = 4


def _make_prop_kernel(n_pad, d, nchunk):
    epw = nchunk * _K
    rows_per_sub = n_pad // _NS
    mesh = plsc.VectorSubcoreMesh(core_axis_name="c", subcore_axis_name="s")

    @functools.partial(
        pl.kernel,
        out_type=jax.ShapeDtypeStruct((_NC, n_pad, d), F32),
        mesh=mesh,
        scratch_types=[
            pltpu.VMEM((_K,), jnp.int32),
            pltpu.VMEM((_K,), jnp.int32),
            pltpu.VMEM((_K, d), F32),
            pltpu.VMEM_SHARED((n_pad, d), F32),
            pltpu.SemaphoreType.DMA,
        ],
    )
    def prop_kernel(hp_hbm, src_hbm, dst_hbm, zeros_hbm, out_hbm,
                    idx_s, idx_d, rows_v, acc, sem):
        c = _core_id()
        s = _subcore_id()
        wid = _worker_id()
        r0 = s * rows_per_sub
        pltpu.sync_copy(zeros_hbm.at[pl.ds(r0, rows_per_sub)],
                        acc.at[pl.ds(r0, rows_per_sub)])
        plsc.subcore_barrier()

        base = wid * epw

        def body(j, _):
            pltpu.sync_copy(src_hbm.at[pl.ds(base + j * _K, _K)], idx_s)
            pltpu.sync_copy(dst_hbm.at[pl.ds(base + j * _K, _K)], idx_d)
            pltpu.async_copy(hp_hbm.at[idx_s], rows_v, sem).wait()
            pltpu.sync_copy(rows_v, acc.at[idx_d], add=True)
            return 0

        lax.fori_loop(0, nchunk, body, 0)
        plsc.subcore_barrier()
        pltpu.sync_copy(acc.at[pl.ds(r0, rows_per_sub)],
                        out_hbm.at[c, pl.ds(r0, rows_per_sub)])

    return prop_kernel


def _dinv_from(degp_ref):
    deg = 1.0 + degp_ref[0, :, 0] + degp_ref[1, :, 0]
    return lax.rsqrt(deg)[:, None]


def _dot(a, b):
    return jnp.dot(a, b, preferred_element_type=F32)


def _row_spec(bm, d):
    return pl.BlockSpec((bm, d), lambda i: (i, 0))


def _part_spec(bm, d):
    return pl.BlockSpec((_NC, bm, d), lambda i: (0, i, 0))


def _full_spec(shape):
    nd = len(shape)
    return pl.BlockSpec(shape, lambda i: (0,) * nd)


def kernel(x, edge_index, W1, b1, W21, b21, W22, b22, W31, b31, W32, b32,
           W33, b33, Wl, bl):
    n, d_in = x.shape
    d_out = W1.shape[1]
    e = edge_index.shape[1]

    src = edge_index[0].astype(jnp.int32)
    dst = edge_index[1].astype(jnp.int32)

    unit = _NW * _K * _IDXD
    e_pad = ((e + unit - 1) // unit) * unit
    n_pad_rows = _pad_nodes(n)
    if e_pad != e:
        pad = e_pad - e
        src = jnp.concatenate([src, jnp.zeros((pad,), jnp.int32)])
        dummy = n + jnp.arange(pad, dtype=jnp.int32) % (n_pad_rows - n)
        dst = jnp.concatenate([dst, dummy])
    nchunk = e_pad // (_NW * _K)
    dst3 = dst.reshape(_NW, nchunk, _K)

    n_pad = n_pad_rows
    zeros_d = jnp.zeros((n_pad, d_out), F32)
    ones_k = jnp.ones((_K, d_out), F32)

    deg_kernel = _make_deg_kernel(n_pad, d_out, nchunk)
    prop_kernel = _make_prop_kernel(n_pad, d_out, nchunk)

    degp = deg_kernel(dst3, ones_k, zeros_d)

    def prop(hp):
        return prop_kernel(hp, src, dst, zeros_d)

    bm = 400
    grid = (n // bm,)

    b1r = b1.reshape(1, -1)
    b21r = b21.reshape(1, -1)
    b22r = b22.reshape(1, -1)
    b31r = b31.reshape(1, -1)
    b32r = b32.reshape(1, -1)
    b33r = b33.reshape(1, -1)
    blr = bl.reshape(1, -1)

    def tc1_body(x_ref, degp_ref, w1_ref, w2_ref, w3_ref,
                 u1_ref, u2_ref, u3_ref):
        dinv = _dinv_from(degp_ref)
        xb = x_ref[...]
        u1_ref[...] = dinv * _dot(xb, w1_ref[...])
        u2_ref[...] = dinv * _dot(xb, w2_ref[...])
        u3_ref[...] = dinv * _dot(xb, w3_ref[...])

    u1p, u2p, u3p = pl.pallas_call(
        tc1_body,
        grid=grid,
        in_specs=[_row_spec(bm, d_in), _part_spec(bm, d_out),
                  _full_spec(W1.shape), _full_spec(W21.shape),
                  _full_spec(W31.shape)],
        out_specs=[_row_spec(bm, d_out)] * 3,
        out_shape=[jax.ShapeDtypeStruct((n, d_out), F32)] * 3,
    )(x, degp, W1, W21, W31)

    p1 = prop(u1p)
    p2 = prop(u2p)
    p3 = prop(u3p)

    def tc2_body(degp_ref, p1_ref, p2_ref, p3_ref, u1_ref, u2_ref, u3_ref,
                 b1_ref, b21_ref, b31_ref, w22_ref, w32_ref,
                 h1_ref, v2_ref, v3_ref):
        dinv = _dinv_from(degp_ref)
        s1 = p1_ref[0] + p1_ref[1] + u1_ref[...]
        h1_ref[...] = jnp.maximum(dinv * s1 + b1_ref[...], 0.0)
        s2 = p2_ref[0] + p2_ref[1] + u2_ref[...]
        t2 = jnp.maximum(dinv * s2 + b21_ref[...], 0.0)
        v2_ref[...] = dinv * _dot(t2, w22_ref[...])
        s3 = p3_ref[0] + p3_ref[1] + u3_ref[...]
        t3 = jnp.maximum(dinv * s3 + b31_ref[...], 0.0)
        v3_ref[...] = dinv * _dot(t3, w32_ref[...])

    h1, v2p, v3p = pl.pallas_call(
        tc2_body,
        grid=grid,
        in_specs=[_part_spec(bm, d_out)] + [_part_spec(bm, d_out)] * 3 +
                 [_row_spec(bm, d_out)] * 3 +
                 [_full_spec((1, d_out))] * 3 +
                 [_full_spec(W22.shape), _full_spec(W32.shape)],
        out_specs=[_row_spec(bm, d_out)] * 3,
        out_shape=[jax.ShapeDtypeStruct((n, d_out), F32)] * 3,
    )(degp, p1, p2, p3, u1p, u2p, u3p, b1r, b21r, b31r, W22, W32)

    q2 = prop(v2p)
    q3 = prop(v3p)

    def tc3_body(degp_ref, q2_ref, q3_ref, v2_ref, v3_ref,
                 b22_ref, b32_ref, w33_ref, h2_ref, z_ref):
        dinv = _dinv_from(degp_ref)
        s2 = q2_ref[0] + q2_ref[1] + v2_ref[...]
        h2_ref[...] = jnp.maximum(dinv * s2 + b22_ref[...], 0.0)
        s3 = q3_ref[0] + q3_ref[1] + v3_ref[...]
        t3 = jnp.maximum(dinv * s3 + b32_ref[...], 0.0)
        z_ref[...] = dinv * _dot(t3, w33_ref[...])

    h2, zp = pl.pallas_call(
        tc3_body,
        grid=grid,
        in_specs=[_part_spec(bm, d_out)] + [_part_spec(bm, d_out)] * 2 +
                 [_row_spec(bm, d_out)] * 2 +
                 [_full_spec((1, d_out))] * 2 + [_full_spec(W33.shape)],
        out_specs=[_row_spec(bm, d_out)] * 2,
        out_shape=[jax.ShapeDtypeStruct((n, d_out), F32)] * 2,
    )(degp, q2, q3, v2p, v3p, b22r, b32r, W33)

    r3 = prop(zp)

    def tc4_body(degp_ref, r3_ref, z_ref, b33_ref, h1_ref, h2_ref, wl_ref,
                 c_ref):
        dinv = _dinv_from(degp_ref)
        s3 = r3_ref[0] + r3_ref[1] + z_ref[...]
        h3 = jnp.maximum(dinv * s3 + b33_ref[...], 0.0)
        wl = wl_ref[...]
        acc = _dot(h1_ref[...], wl[0:d_out])
        acc = acc + _dot(h2_ref[...], wl[d_out:2 * d_out])
        acc = acc + _dot(h3, wl[2 * d_out:3 * d_out])
        c_ref[...] = dinv * acc

    cp = pl.pallas_call(
        tc4_body,
        grid=grid,
        in_specs=[_part_spec(bm, d_out), _part_spec(bm, d_out),
                  _row_spec(bm, d_out), _full_spec((1, d_out)),
                  _row_spec(bm, d_out), _row_spec(bm, d_out),
                  _full_spec(Wl.shape)],
        out_specs=_row_spec(bm, d_out),
        out_shape=jax.ShapeDtypeStruct((n, d_out), F32),
    )(degp, r3, zp, b33r, h1, h2, Wl)

    pc = prop(cp)

    def tc5_body(degp_ref, pc_ref, c_ref, bl_ref, out_ref):
        dinv = _dinv_from(degp_ref)
        s = pc_ref[0] + pc_ref[1] + c_ref[...]
        out_ref[...] = dinv * s + bl_ref[...]

    out = pl.pallas_call(
        tc5_body,
        grid=grid,
        in_specs=[_part_spec(bm, d_out), _part_spec(bm, d_out),
                  _row_spec(bm, d_out), _full_spec((1, d_out))],
        out_specs=_row_spec(bm, d_out),
        out_shape=jax.ShapeDtypeStruct((n, d_out), F32),
    )(degp, pc, cp, blr)

    return out


# speedup vs baseline: 1.4699x; 1.4699x over previous
import functools

import jax
import jax.numpy as jnp
from jax import lax
from jax.experimental import pallas as pl
from jax.experimental.pallas import tpu as pltpu
from jax.experimental.pallas import tpu_sc as plsc

F32 = jnp.float32

_NC = 2
_NS = 16
_NW = _NC * _NS
_K = 128


def _pad_nodes(n):
    unit = _NS * 8
    return ((n + 1 + unit - 1) // unit) * unit


def _worker_id():
    return lax.axis_index("s") * _NC + lax.axis_index("c")


def _core_id():
    return lax.axis_index("c")


def _subcore_id():
    return lax.axis_index("s")


def _make_deg_kernel(n_pad, d, nchunk):
    epw = nchunk * _K
    rows_per_sub = n_pad // _NS
    mesh = plsc.VectorSubcoreMesh(core_axis_name="c", subcore_axis_name="s")

    @functools.partial(
        pl.kernel,
        out_type=jax.ShapeDtypeStruct((_NC, n_pad, d), F32),
        mesh=mesh,
        scratch_types=[
            pltpu.VMEM((_K,), jnp.int32),
            pltpu.VMEM((_K, d), F32),
            pltpu.VMEM_SHARED((n_pad, d), F32),
        ],
    )
    def deg_kernel(dst_hbm, ones_hbm, zeros_hbm, out_hbm, idx_v, ones_v,
                   acc):
        c = _core_id()
        s = _subcore_id()
        wid = _worker_id()
        r0 = s * rows_per_sub
        pltpu.sync_copy(zeros_hbm.at[pl.ds(r0, rows_per_sub)],
                        acc.at[pl.ds(r0, rows_per_sub)])
        pltpu.sync_copy(ones_hbm, ones_v)
        plsc.subcore_barrier()

        base = wid * epw

        def body(j, _):
            pltpu.sync_copy(dst_hbm.at[pl.ds(base + j * _K, _K)], idx_v)
            pltpu.sync_copy(ones_v, acc.at[idx_v], add=True)
            return 0

        lax.fori_loop(0, nchunk, body, 0)
        plsc.subcore_barrier()
        pltpu.sync_copy(acc.at[pl.ds(r0, rows_per_sub)],
                        out_hbm.at[c, pl.ds(r0, rows_per_sub)])

    return deg_kernel


_IDXD = 4


def _make_prop_kernel(n_pad, d, nchunk):
    epw = nchunk * _K
    rows_per_sub = n_pad // _NS
    mesh = plsc.VectorSubcoreMesh(core_axis_name="c", subcore_axis_name="s")

    @functools.partial(
        pl.kernel,
        out_type=jax.ShapeDtypeStruct((_NC, n_pad, d), F32),
        mesh=mesh,
        scratch_types=[
            pltpu.VMEM((_K,), jnp.int32),
            pltpu.VMEM((_K,), jnp.int32),
            pltpu.VMEM((_K, d), F32),
            pltpu.VMEM_SHARED((n_pad, d), F32),
            pltpu.SemaphoreType.DMA,
        ],
    )
    def prop_kernel(hp_hbm, src_hbm, dst_hbm, zeros_hbm, out_hbm,
                    idx_s, idx_d, rows_v, acc, sem):
        c = _core_id()
        s = _subcore_id()
        wid = _worker_id()
        r0 = s * rows_per_sub
        pltpu.sync_copy(zeros_hbm.at[pl.ds(r0, rows_per_sub)],
                        acc.at[pl.ds(r0, rows_per_sub)])
        plsc.subcore_barrier()

        base = wid * epw

        def body(j, _):
            pltpu.sync_copy(src_hbm.at[pl.ds(base + j * _K, _K)], idx_s)
            pltpu.sync_copy(dst_hbm.at[pl.ds(base + j * _K, _K)], idx_d)
            pltpu.async_copy(hp_hbm.at[idx_s], rows_v, sem).wait()
            pltpu.sync_copy(rows_v, acc.at[idx_d], add=True)
            return 0

        lax.fori_loop(0, nchunk, body, 0)
        plsc.subcore_barrier()
        pltpu.sync_copy(acc.at[pl.ds(r0, rows_per_sub)],
                        out_hbm.at[c, pl.ds(r0, rows_per_sub)])

    return prop_kernel


def _dinv_from(degp_ref):
    deg = 1.0 + degp_ref[0, :, 0] + degp_ref[1, :, 0]
    return lax.rsqrt(deg)[:, None]


def _dot(a, b):
    return jnp.dot(a, b, preferred_element_type=F32)


def _row_spec(bm, d):
    return pl.BlockSpec((bm, d), lambda i: (i, 0))


def _part_spec(bm, d):
    return pl.BlockSpec((_NC, bm, d), lambda i: (0, i, 0))


def _full_spec(shape):
    nd = len(shape)
    return pl.BlockSpec(shape, lambda i: (0,) * nd)


def kernel(x, edge_index, W1, b1, W21, b21, W22, b22, W31, b31, W32, b32,
           W33, b33, Wl, bl):
    n, d_in = x.shape
    d_out = W1.shape[1]
    e = edge_index.shape[1]

    src = edge_index[0].astype(jnp.int32)
    dst = edge_index[1].astype(jnp.int32)

    unit = _NW * _K
    e_pad = ((e + unit - 1) // unit) * unit
    n_pad_rows = _pad_nodes(n)
    if e_pad != e:
        pad = e_pad - e
        src = jnp.concatenate([src, jnp.zeros((pad,), jnp.int32)])
        dst = jnp.concatenate([dst, jnp.full((pad,), n, jnp.int32)])
    nchunk = e_pad // (_NW * _K)

    n_pad = n_pad_rows
    zeros_d = jnp.zeros((n_pad, d_out), F32)
    ones_k = jnp.ones((_K, d_out), F32)

    deg_kernel = _make_deg_kernel(n_pad, d_out, nchunk)
    prop_kernel = _make_prop_kernel(n_pad, d_out, nchunk)

    degp = deg_kernel(dst, ones_k, zeros_d)

    def prop(hp):
        return prop_kernel(hp, src, dst, zeros_d)

    bm = 400
    grid = (n // bm,)

    b1r = b1.reshape(1, -1)
    b21r = b21.reshape(1, -1)
    b22r = b22.reshape(1, -1)
    b31r = b31.reshape(1, -1)
    b32r = b32.reshape(1, -1)
    b33r = b33.reshape(1, -1)
    blr = bl.reshape(1, -1)

    def tc1_body(x_ref, degp_ref, w1_ref, w2_ref, w3_ref,
                 u1_ref, u2_ref, u3_ref):
        dinv = _dinv_from(degp_ref)
        xb = x_ref[...]
        u1_ref[...] = dinv * _dot(xb, w1_ref[...])
        u2_ref[...] = dinv * _dot(xb, w2_ref[...])
        u3_ref[...] = dinv * _dot(xb, w3_ref[...])

    u1p, u2p, u3p = pl.pallas_call(
        tc1_body,
        grid=grid,
        in_specs=[_row_spec(bm, d_in), _part_spec(bm, d_out),
                  _full_spec(W1.shape), _full_spec(W21.shape),
                  _full_spec(W31.shape)],
        out_specs=[_row_spec(bm, d_out)] * 3,
        out_shape=[jax.ShapeDtypeStruct((n, d_out), F32)] * 3,
    )(x, degp, W1, W21, W31)

    p1 = prop(u1p)
    p2 = prop(u2p)
    p3 = prop(u3p)

    def tc2_body(degp_ref, p1_ref, p2_ref, p3_ref, u1_ref, u2_ref, u3_ref,
                 b1_ref, b21_ref, b31_ref, w22_ref, w32_ref,
                 h1_ref, v2_ref, v3_ref):
        dinv = _dinv_from(degp_ref)
        s1 = p1_ref[0] + p1_ref[1] + u1_ref[...]
        h1_ref[...] = jnp.maximum(dinv * s1 + b1_ref[...], 0.0)
        s2 = p2_ref[0] + p2_ref[1] + u2_ref[...]
        t2 = jnp.maximum(dinv * s2 + b21_ref[...], 0.0)
        v2_ref[...] = dinv * _dot(t2, w22_ref[...])
        s3 = p3_ref[0] + p3_ref[1] + u3_ref[...]
        t3 = jnp.maximum(dinv * s3 + b31_ref[...], 0.0)
        v3_ref[...] = dinv * _dot(t3, w32_ref[...])

    h1, v2p, v3p = pl.pallas_call(
        tc2_body,
        grid=grid,
        in_specs=[_part_spec(bm, d_out)] + [_part_spec(bm, d_out)] * 3 +
                 [_row_spec(bm, d_out)] * 3 +
                 [_full_spec((1, d_out))] * 3 +
                 [_full_spec(W22.shape), _full_spec(W32.shape)],
        out_specs=[_row_spec(bm, d_out)] * 3,
        out_shape=[jax.ShapeDtypeStruct((n, d_out), F32)] * 3,
    )(degp, p1, p2, p3, u1p, u2p, u3p, b1r, b21r, b31r, W22, W32)

    q2 = prop(v2p)
    q3 = prop(v3p)

    def tc3_body(degp_ref, q2_ref, q3_ref, v2_ref, v3_ref,
                 b22_ref, b32_ref, w33_ref, h2_ref, z_ref):
        dinv = _dinv_from(degp_ref)
        s2 = q2_ref[0] + q2_ref[1] + v2_ref[...]
        h2_ref[...] = jnp.maximum(dinv * s2 + b22_ref[...], 0.0)
        s3 = q3_ref[0] + q3_ref[1] + v3_ref[...]
        t3 = jnp.maximum(dinv * s3 + b32_ref[...], 0.0)
        z_ref[...] = dinv * _dot(t3, w33_ref[...])

    h2, zp = pl.pallas_call(
        tc3_body,
        grid=grid,
        in_specs=[_part_spec(bm, d_out)] + [_part_spec(bm, d_out)] * 2 +
                 [_row_spec(bm, d_out)] * 2 +
                 [_full_spec((1, d_out))] * 2 + [_full_spec(W33.shape)],
        out_specs=[_row_spec(bm, d_out)] * 2,
        out_shape=[jax.ShapeDtypeStruct((n, d_out), F32)] * 2,
    )(degp, q2, q3, v2p, v3p, b22r, b32r, W33)

    r3 = prop(zp)

    def tc4_body(degp_ref, r3_ref, z_ref, b33_ref, h1_ref, h2_ref, wl_ref,
                 c_ref):
        dinv = _dinv_from(degp_ref)
        s3 = r3_ref[0] + r3_ref[1] + z_ref[...]
        h3 = jnp.maximum(dinv * s3 + b33_ref[...], 0.0)
        wl = wl_ref[...]
        acc = _dot(h1_ref[...], wl[0:d_out])
        acc = acc + _dot(h2_ref[...], wl[d_out:2 * d_out])
        acc = acc + _dot(h3, wl[2 * d_out:3 * d_out])
        c_ref[...] = dinv * acc

    cp = pl.pallas_call(
        tc4_body,
        grid=grid,
        in_specs=[_part_spec(bm, d_out), _part_spec(bm, d_out),
                  _row_spec(bm, d_out), _full_spec((1, d_out)),
                  _row_spec(bm, d_out), _row_spec(bm, d_out),
                  _full_spec(Wl.shape)],
        out_specs=_row_spec(bm, d_out),
        out_shape=jax.ShapeDtypeStruct((n, d_out), F32),
    )(degp, r3, zp, b33r, h1, h2, Wl)

    pc = prop(cp)

    def tc5_body(degp_ref, pc_ref, c_ref, bl_ref, out_ref):
        dinv = _dinv_from(degp_ref)
        s = pc_ref[0] + pc_ref[1] + c_ref[...]
        out_ref[...] = dinv * s + bl_ref[...]

    out = pl.pallas_call(
        tc5_body,
        grid=grid,
        in_specs=[_part_spec(bm, d_out), _part_spec(bm, d_out),
                  _row_spec(bm, d_out), _full_spec((1, d_out))],
        out_specs=_row_spec(bm, d_out),
        out_shape=jax.ShapeDtypeStruct((n, d_out), F32),
    )(degp, pc, cp, blr)

    return out
